# Initial kernel scaffold; baseline (speedup 1.0000x reference)
import jax
import jax.numpy as jnp
from jax.experimental import pallas as pl


def kernel(pos_encoding, t):
    raise NotImplementedError("write your pallas kernel here")


# speedup vs baseline: 2.2587x; 2.2587x over previous
import functools

import jax
import jax.numpy as jnp
from jax import lax
from jax.experimental import pallas as pl
from jax.experimental.pallas import tpu as pltpu
from jax.experimental.pallas import tpu_sc as plsc

_MAX_LEN = 1000
_D = 128
_B = 16384
_NC = 2
_NS = 16
_NW = _NC * _NS
_BPW = _B // _NW
_CHUNK = 128
_NCHUNK = _BPW // _CHUNK


def _make_gather():
    mesh = plsc.VectorSubcoreMesh(core_axis_name="c", subcore_axis_name="s")

    @functools.partial(
        pl.kernel,
        mesh=mesh,
        out_type=jax.ShapeDtypeStruct((_NW, _BPW, _D), jnp.float32),
        scratch_types=[
            pltpu.VMEM((_NCHUNK, _CHUNK), jnp.int32),
            pltpu.VMEM((_BPW, _D), jnp.float32),
            pltpu.SemaphoreType.DMA,
        ],
    )
    def gather_kernel(table_hbm, idx_hbm, out_hbm, idx_v, rows_v, sem):
        wid = lax.axis_index("s") * _NC + lax.axis_index("c")
        pltpu.sync_copy(idx_hbm.at[wid], idx_v)
        copies = []
        for j in range(_NCHUNK):
            copies.append(
                pltpu.make_async_copy(
                    table_hbm.at[idx_v.at[j]],
                    rows_v.at[pl.ds(j * _CHUNK, _CHUNK)],
                    sem,
                )
            )
        for c in copies:
            c.start()
        for c in copies:
            c.wait()
        pltpu.sync_copy(rows_v, out_hbm.at[wid])

    return gather_kernel


_gather = _make_gather()


def kernel(pos_encoding, t):
    idx = jnp.reshape(t.astype(jnp.int32), (_NW, _NCHUNK, _CHUNK))
    out = _gather(pos_encoding, idx)
    return jnp.reshape(out, (_B, _D))


# speedup vs baseline: 2.2757x; 1.0075x over previous
import functools

import jax
import jax.numpy as jnp
from jax import lax
from jax.experimental import pallas as pl
from jax.experimental.pallas import tpu as pltpu
from jax.experimental.pallas import tpu_sc as plsc

_MAX_LEN = 1000
_D = 128
_B = 16384
_NC = 2
_NS = 16
_NW = _NC * _NS
_BPW = _B // _NW
_CHUNK = 128
_NCHUNK = _BPW // _CHUNK


def _make_gather():
    mesh = plsc.VectorSubcoreMesh(core_axis_name="c", subcore_axis_name="s")

    @functools.partial(
        pl.kernel,
        mesh=mesh,
        out_type=jax.ShapeDtypeStruct((_NW, _BPW, _D), jnp.float32),
        scratch_types=[
            pltpu.VMEM((_NCHUNK, _CHUNK), jnp.int32),
            pltpu.VMEM((_BPW, _D), jnp.float32),
            *([pltpu.SemaphoreType.DMA] * _NCHUNK),
            pltpu.SemaphoreType.DMA,
        ],
    )
    def gather_kernel(table_hbm, idx_hbm, out_hbm, idx_v, rows_v, *sems):
        gsems, ssem = sems[:_NCHUNK], sems[_NCHUNK]
        wid = lax.axis_index("s") * _NC + lax.axis_index("c")
        pltpu.sync_copy(idx_hbm.at[wid], idx_v)
        gathers = []
        for j in range(_NCHUNK):
            gathers.append(
                pltpu.make_async_copy(
                    table_hbm.at[idx_v.at[j]],
                    rows_v.at[pl.ds(j * _CHUNK, _CHUNK)],
                    gsems[j],
                )
            )
        for c in gathers:
            c.start()
        stores = []
        for j in range(_NCHUNK):
            gathers[j].wait()
            s = pltpu.make_async_copy(
                rows_v.at[pl.ds(j * _CHUNK, _CHUNK)],
                out_hbm.at[wid, pl.ds(j * _CHUNK, _CHUNK)],
                ssem,
            )
            s.start()
            stores.append(s)
        for s in stores:
            s.wait()

    return gather_kernel


_gather = _make_gather()


def kernel(pos_encoding, t):
    idx = jnp.reshape(t.astype(jnp.int32), (_NW, _NCHUNK, _CHUNK))
    out = _gather(pos_encoding, idx)
    return jnp.reshape(out, (_B, _D))


# speedup vs baseline: 2.7394x; 1.2038x over previous
import functools

import jax
import jax.numpy as jnp
from jax import lax
from jax.experimental import pallas as pl
from jax.experimental.pallas import tpu as pltpu
from jax.experimental.pallas import tpu_sc as plsc

_MAX_LEN = 1000
_D = 128
_B = 16384
_NC = 2
_NS = 16
_NW = _NC * _NS
_BPW = _B // _NW
_CHUNK = 128
_NCHUNK = _BPW // _CHUNK
_TROWS = 64
_TREM = _MAX_LEN - _TROWS * (_NS - 1)


def _make_gather():
    mesh = plsc.VectorSubcoreMesh(core_axis_name="c", subcore_axis_name="s")

    @functools.partial(
        pl.kernel,
        mesh=mesh,
        out_type=jax.ShapeDtypeStruct((_NW, _BPW, _D), jnp.float32),
        scratch_types=[
            pltpu.VMEM_SHARED((_MAX_LEN, _D), jnp.float32),
            pltpu.VMEM((_NCHUNK, _CHUNK), jnp.int32),
            pltpu.VMEM((_BPW, _D), jnp.float32),
            *([pltpu.SemaphoreType.DMA] * _NCHUNK),
            pltpu.SemaphoreType.DMA,
        ],
    )
    def gather_kernel(table_hbm, idx_hbm, out_hbm, table_sp, idx_v, rows_v, *sems):
        gsems, ssem = sems[:_NCHUNK], sems[_NCHUNK]
        cid = lax.axis_index("c")
        sid = lax.axis_index("s")
        wid = sid * _NC + cid
        pltpu.sync_copy(idx_hbm.at[wid], idx_v)
        @pl.when(sid < _NS - 1)
        def _():
            pltpu.sync_copy(
                table_hbm.at[pl.ds(sid * _TROWS, _TROWS)],
                table_sp.at[pl.ds(sid * _TROWS, _TROWS)],
            )

        @pl.when(sid == _NS - 1)
        def _():
            pltpu.sync_copy(
                table_hbm.at[pl.ds((_NS - 1) * _TROWS, _TREM)],
                table_sp.at[pl.ds((_NS - 1) * _TROWS, _TREM)],
            )

        plsc.subcore_barrier()
        gathers = []
        for j in range(_NCHUNK):
            gathers.append(
                pltpu.make_async_copy(
                    table_sp.at[idx_v.at[j]],
                    rows_v.at[pl.ds(j * _CHUNK, _CHUNK)],
                    gsems[j],
                )
            )
        for c in gathers:
            c.start()
        stores = []
        for j in range(_NCHUNK):
            gathers[j].wait()
            s = pltpu.make_async_copy(
                rows_v.at[pl.ds(j * _CHUNK, _CHUNK)],
                out_hbm.at[wid, pl.ds(j * _CHUNK, _CHUNK)],
                ssem,
            )
            s.start()
            stores.append(s)
        for s in stores:
            s.wait()

    return gather_kernel


_gather = _make_gather()


def kernel(pos_encoding, t):
    idx = jnp.reshape(t.astype(jnp.int32), (_NW, _NCHUNK, _CHUNK))
    out = _gather(pos_encoding, idx)
    return jnp.reshape(out, (_B, _D))


# speedup vs baseline: 2.7921x; 1.0193x over previous
import functools

import jax
import jax.numpy as jnp
from jax import lax
from jax.experimental import pallas as pl
from jax.experimental.pallas import tpu as pltpu
from jax.experimental.pallas import tpu_sc as plsc

_MAX_LEN = 1000
_D = 128
_B = 16384
_NC = 2
_NS = 16
_NW = _NC * _NS
_BPW = _B // _NW
_CHUNK = 128
_NCHUNK = _BPW // _CHUNK
_TROWS = 64
_TREM = _MAX_LEN - _TROWS * (_NS - 1)


def _make_gather():
    mesh = plsc.VectorSubcoreMesh(core_axis_name="c", subcore_axis_name="s")

    @functools.partial(
        pl.kernel,
        mesh=mesh,
        out_type=jax.ShapeDtypeStruct((_NW, _BPW, _D), jnp.float32),
        scratch_types=[
            pltpu.VMEM_SHARED((_MAX_LEN, _D), jnp.float32),
            pltpu.VMEM((_NCHUNK, _CHUNK), jnp.int32),
            pltpu.VMEM((_BPW, _D), jnp.float32),
            *([pltpu.SemaphoreType.DMA] * _NCHUNK),
            pltpu.SemaphoreType.DMA,
        ],
    )
    def gather_kernel(table_hbm, idx_hbm, out_hbm, table_sp, idx_v, rows_v, *sems):
        gsems, ssem = sems[:_NCHUNK], sems[_NCHUNK]
        cid = lax.axis_index("c")
        sid = lax.axis_index("s")
        wid = sid * _NC + cid
        idx_cp = pltpu.make_async_copy(idx_hbm.at[wid], idx_v, ssem)
        idx_cp.start()
        @pl.when(sid < _NS - 1)
        def _():
            pltpu.sync_copy(
                table_hbm.at[pl.ds(sid * _TROWS, _TROWS)],
                table_sp.at[pl.ds(sid * _TROWS, _TROWS)],
            )

        @pl.when(sid == _NS - 1)
        def _():
            pltpu.sync_copy(
                table_hbm.at[pl.ds((_NS - 1) * _TROWS, _TREM)],
                table_sp.at[pl.ds((_NS - 1) * _TROWS, _TREM)],
            )

        idx_cp.wait()
        plsc.subcore_barrier()
        gathers = []
        for j in range(_NCHUNK):
            gathers.append(
                pltpu.make_async_copy(
                    table_sp.at[idx_v.at[j]],
                    rows_v.at[pl.ds(j * _CHUNK, _CHUNK)],
                    gsems[j],
                )
            )
        for c in gathers:
            c.start()
        stores = []
        for j in range(_NCHUNK):
            gathers[j].wait()
            s = pltpu.make_async_copy(
                rows_v.at[pl.ds(j * _CHUNK, _CHUNK)],
                out_hbm.at[wid, pl.ds(j * _CHUNK, _CHUNK)],
                ssem,
            )
            s.start()
            stores.append(s)
        for s in stores:
            s.wait()

    return gather_kernel


_gather = _make_gather()


def kernel(pos_encoding, t):
    idx = jnp.reshape(t.astype(jnp.int32), (_NW, _NCHUNK, _CHUNK))
    out = _gather(pos_encoding, idx)
    return jnp.reshape(out, (_B, _D))


# speedup vs baseline: 2.8035x; 1.0041x over previous
import functools

import jax
import jax.numpy as jnp
from jax import lax
from jax.experimental import pallas as pl
from jax.experimental.pallas import tpu as pltpu
from jax.experimental.pallas import tpu_sc as plsc

_MAX_LEN = 1000
_D = 128
_B = 16384
_NC = 2
_NS = 16
_NW = _NC * _NS
_BPW = _B // _NW
_CHUNK = 64
_NCHUNK = _BPW // _CHUNK
_TROWS = 64
_TREM = _MAX_LEN - _TROWS * (_NS - 1)


def _make_gather():
    mesh = plsc.VectorSubcoreMesh(core_axis_name="c", subcore_axis_name="s")

    @functools.partial(
        pl.kernel,
        mesh=mesh,
        out_type=jax.ShapeDtypeStruct((_NW, _BPW, _D), jnp.float32),
        scratch_types=[
            pltpu.VMEM_SHARED((_MAX_LEN, _D), jnp.float32),
            pltpu.VMEM((_NCHUNK, _CHUNK), jnp.int32),
            pltpu.VMEM((_BPW, _D), jnp.float32),
            *([pltpu.SemaphoreType.DMA] * _NCHUNK),
            pltpu.SemaphoreType.DMA,
        ],
    )
    def gather_kernel(table_hbm, idx_hbm, out_hbm, table_sp, idx_v, rows_v, *sems):
        gsems, ssem = sems[:_NCHUNK], sems[_NCHUNK]
        cid = lax.axis_index("c")
        sid = lax.axis_index("s")
        wid = sid * _NC + cid
        idx_cp = pltpu.make_async_copy(idx_hbm.at[wid], idx_v, ssem)
        idx_cp.start()
        @pl.when(sid < _NS - 1)
        def _():
            pltpu.sync_copy(
                table_hbm.at[pl.ds(sid * _TROWS, _TROWS)],
                table_sp.at[pl.ds(sid * _TROWS, _TROWS)],
            )

        @pl.when(sid == _NS - 1)
        def _():
            pltpu.sync_copy(
                table_hbm.at[pl.ds((_NS - 1) * _TROWS, _TREM)],
                table_sp.at[pl.ds((_NS - 1) * _TROWS, _TREM)],
            )

        idx_cp.wait()
        plsc.subcore_barrier()
        gathers = []
        for j in range(_NCHUNK):
            gathers.append(
                pltpu.make_async_copy(
                    table_sp.at[idx_v.at[j]],
                    rows_v.at[pl.ds(j * _CHUNK, _CHUNK)],
                    gsems[j],
                )
            )
        for c in gathers:
            c.start()
        stores = []
        for j in range(_NCHUNK):
            gathers[j].wait()
            s = pltpu.make_async_copy(
                rows_v.at[pl.ds(j * _CHUNK, _CHUNK)],
                out_hbm.at[wid, pl.ds(j * _CHUNK, _CHUNK)],
                ssem,
            )
            s.start()
            stores.append(s)
        for s in stores:
            s.wait()

    return gather_kernel


_gather = _make_gather()


def kernel(pos_encoding, t):
    idx = jnp.reshape(t.astype(jnp.int32), (_NW, _NCHUNK, _CHUNK))
    out = _gather(pos_encoding, idx)
    return jnp.reshape(out, (_B, _D))
